# baseline (speedup 1.0000x reference)
import functools

import jax
import jax.numpy as jnp
from jax import lax
from jax.experimental import pallas as pl
from jax.experimental.pallas import tpu as pltpu
from jax.experimental.pallas import tpu_sc as plsc

N_NODES = 10000
D_FEAT = 128
N_EDGES = 320000

NC = 2
NS = 16
NW = NC * NS

CHUNK = 128
RING = 3
IR = 4
CHUNKS_PER_W = 81
ROUNDS = CHUNKS_PER_W // RING
PER_W = CHUNK * CHUNKS_PER_W
E_PAD = PER_W * NW

ACC_ROWS = N_NODES
ZERO_TILES = 10
ROWS_PER_ZT = ACC_ROWS // ZERO_TILES

_mesh = plsc.VectorSubcoreMesh(core_axis_name="c", subcore_axis_name="s")


@functools.partial(
    pl.kernel,
    mesh=_mesh,
    out_type=jax.ShapeDtypeStruct((NC, ACC_ROWS, D_FEAT), jnp.float32),
    scratch_types=[
        pltpu.VMEM_SHARED((ACC_ROWS, D_FEAT), jnp.float32),
        pltpu.VMEM((IR, 2, CHUNK), jnp.int32),
        pltpu.VMEM((RING, CHUNK, D_FEAT), jnp.float32),
        pltpu.SemaphoreType.DMA,
        pltpu.SemaphoreType.DMA,
        pltpu.SemaphoreType.DMA,
    ],
)
def _sc_gather_scatter(x_hbm, idx_hbm, zeros_hbm, part_hbm,
                       acc, idx_v, rows_v, isem, gsem, ssem):
    c = lax.axis_index("c")
    s = lax.axis_index("s")
    wid = s * NC + c

    for b in range(RING - 1):
        pltpu.sync_copy(idx_hbm.at[wid, b], idx_v.at[b])
        pltpu.async_copy(x_hbm.at[idx_v.at[b, 0]], rows_v.at[b], gsem)
    pltpu.async_copy(idx_hbm.at[wid, RING - 1], idx_v.at[RING - 1], isem)

    @pl.when(s < ZERO_TILES)
    def _():
        pltpu.sync_copy(zeros_hbm.at[pl.ds(s * ROWS_PER_ZT, ROWS_PER_ZT)],
                        acc.at[pl.ds(s * ROWS_PER_ZT, ROWS_PER_ZT)])
    plsc.subcore_barrier()

    def round_body(r, carry):
        for b in range(RING):
            g = r * RING + b
            pltpu.make_async_copy(x_hbm.at[idx_v.at[g % IR, 0]],
                                  rows_v.at[b], gsem).wait()
            pltpu.async_copy(rows_v.at[b], acc.at[idx_v.at[g % IR, 1]], ssem,
                             add=True)
            bp = (b - 1) % RING
            drain = pltpu.make_async_copy(
                rows_v.at[bp], acc.at[idx_v.at[(g - 1) % IR, 1]], ssem)
            if b == 0:
                @pl.when(r > 0)
                def _():
                    drain.wait()
            else:
                drain.wait()

            g2 = g + RING - 1
            g3 = g + RING

            def stage():
                pltpu.make_async_copy(idx_hbm.at[wid, g2],
                                      idx_v.at[g2 % IR], isem).wait()
                pltpu.async_copy(x_hbm.at[idx_v.at[g2 % IR, 0]],
                                 rows_v.at[bp], gsem)

            def prefetch():
                pltpu.async_copy(idx_hbm.at[wid, g3], idx_v.at[g3 % IR], isem)

            if (ROUNDS - 1) * RING + b + RING - 1 < CHUNKS_PER_W:
                stage()
            else:
                pl.when(g2 < CHUNKS_PER_W)(stage)
            if (ROUNDS - 1) * RING + b + RING < CHUNKS_PER_W:
                prefetch()
            else:
                pl.when(g3 < CHUNKS_PER_W)(prefetch)
        return carry

    lax.fori_loop(0, ROUNDS, round_body, 0)

    pltpu.make_async_copy(rows_v.at[RING - 1],
                          acc.at[idx_v.at[(CHUNKS_PER_W - 1) % IR, 1]],
                          ssem).wait()

    plsc.subcore_barrier()
    @pl.when(s < ZERO_TILES)
    def _():
        pltpu.sync_copy(acc.at[pl.ds(s * ROWS_PER_ZT, ROWS_PER_ZT)],
                        part_hbm.at[c, pl.ds(s * ROWS_PER_ZT, ROWS_PER_ZT)])


def _add_body(a_ref, b_ref, o_ref):
    o_ref[...] = a_ref[0] + b_ref[0]


_ADD_ROWS = 400


def _combine_partials(part):
    return pl.pallas_call(
        _add_body,
        out_shape=jax.ShapeDtypeStruct((N_NODES, D_FEAT), jnp.float32),
        grid=(N_NODES // _ADD_ROWS,),
        in_specs=[
            pl.BlockSpec((1, _ADD_ROWS, D_FEAT), lambda i: (0, i, 0)),
            pl.BlockSpec((1, _ADD_ROWS, D_FEAT), lambda i: (1, i, 0)),
        ],
        out_specs=pl.BlockSpec((_ADD_ROWS, D_FEAT), lambda i: (i, 0)),
    )(part, part)


def kernel(x, edge_index):
    x1 = jnp.concatenate([x, jnp.zeros((1, D_FEAT), jnp.float32)])
    pad = E_PAD - N_EDGES
    src = jnp.concatenate(
        [edge_index[0], jnp.full((pad,), N_NODES, jnp.int32)])
    dst = jnp.concatenate([edge_index[1], jnp.zeros((pad,), jnp.int32)])
    idx = jnp.stack([src.reshape(NW, CHUNKS_PER_W, CHUNK),
                     dst.reshape(NW, CHUNKS_PER_W, CHUNK)], axis=2)
    zeros = jnp.zeros((ACC_ROWS, D_FEAT), jnp.float32)
    part = _sc_gather_scatter(x1, idx, zeros)
    return _combine_partials(part)


# speedup vs baseline: 2.7421x; 2.7421x over previous
import functools

import jax
import jax.numpy as jnp
from jax import lax
from jax.experimental import pallas as pl
from jax.experimental.pallas import tpu as pltpu
from jax.experimental.pallas import tpu_sc as plsc

N_NODES = 10000
D_FEAT = 128
N_EDGES = 320000

NC = 2
NS = 16
NW = NC * NS

CHUNK = 112
RING = 3
IR = 4
CHUNKS_PER_W = 90
ROUNDS = CHUNKS_PER_W // RING
PER_W = CHUNK * CHUNKS_PER_W
E_PAD = PER_W * NW

ACC_ROWS = N_NODES
ZERO_TILES = 10
ROWS_PER_ZT = ACC_ROWS // ZERO_TILES

_mesh = plsc.VectorSubcoreMesh(core_axis_name="c", subcore_axis_name="s")


@functools.partial(
    pl.kernel,
    mesh=_mesh,
    out_type=jax.ShapeDtypeStruct((NC, ACC_ROWS, D_FEAT), jnp.float32),
    scratch_types=[
        pltpu.VMEM_SHARED((ACC_ROWS, D_FEAT), jnp.float32),
        pltpu.VMEM((IR, 2, CHUNK), jnp.int32),
        pltpu.VMEM((RING, CHUNK, D_FEAT), jnp.float32),
        pltpu.SemaphoreType.DMA,
        pltpu.SemaphoreType.DMA,
        pltpu.SemaphoreType.DMA,
    ],
)
def _sc_gather_scatter(x_hbm, idx_hbm, zeros_hbm, part_hbm,
                       acc, idx_v, rows_v, isem, gsem, ssem):
    c = lax.axis_index("c")
    s = lax.axis_index("s")
    wid = s * NC + c

    for b in range(RING - 1):
        pltpu.sync_copy(idx_hbm.at[wid, b], idx_v.at[b])
        pltpu.async_copy(x_hbm.at[idx_v.at[b, 0]], rows_v.at[b], gsem)
    pltpu.async_copy(idx_hbm.at[wid, RING - 1], idx_v.at[RING - 1], isem)

    @pl.when(s < ZERO_TILES)
    def _():
        pltpu.sync_copy(zeros_hbm.at[pl.ds(s * ROWS_PER_ZT, ROWS_PER_ZT)],
                        acc.at[pl.ds(s * ROWS_PER_ZT, ROWS_PER_ZT)])
    plsc.subcore_barrier()

    def round_body(r, carry):
        for b in range(RING):
            g = r * RING + b
            pltpu.make_async_copy(x_hbm.at[idx_v.at[g % IR, 0]],
                                  rows_v.at[b], gsem).wait()
            pltpu.async_copy(rows_v.at[b], acc.at[idx_v.at[g % IR, 1]], ssem,
                             add=True)
            bp = (b - 1) % RING
            drain = pltpu.make_async_copy(
                rows_v.at[bp], acc.at[idx_v.at[(g - 1) % IR, 1]], ssem)
            if b == 0:
                @pl.when(r > 0)
                def _():
                    drain.wait()
            else:
                drain.wait()

            g2 = g + RING - 1
            g3 = g + RING

            def stage():
                pltpu.make_async_copy(idx_hbm.at[wid, g2],
                                      idx_v.at[g2 % IR], isem).wait()
                pltpu.async_copy(x_hbm.at[idx_v.at[g2 % IR, 0]],
                                 rows_v.at[bp], gsem)

            def prefetch():
                pltpu.async_copy(idx_hbm.at[wid, g3], idx_v.at[g3 % IR], isem)

            if (ROUNDS - 1) * RING + b + RING - 1 < CHUNKS_PER_W:
                stage()
            else:
                pl.when(g2 < CHUNKS_PER_W)(stage)
            if (ROUNDS - 1) * RING + b + RING < CHUNKS_PER_W:
                prefetch()
            else:
                pl.when(g3 < CHUNKS_PER_W)(prefetch)
        return carry

    lax.fori_loop(0, ROUNDS, round_body, 0)

    pltpu.make_async_copy(rows_v.at[RING - 1],
                          acc.at[idx_v.at[(CHUNKS_PER_W - 1) % IR, 1]],
                          ssem).wait()

    plsc.subcore_barrier()
    @pl.when(s < ZERO_TILES)
    def _():
        pltpu.sync_copy(acc.at[pl.ds(s * ROWS_PER_ZT, ROWS_PER_ZT)],
                        part_hbm.at[c, pl.ds(s * ROWS_PER_ZT, ROWS_PER_ZT)])


def _add_body(a_ref, b_ref, o_ref):
    o_ref[...] = a_ref[0] + b_ref[0]


_ADD_ROWS = 400


def _combine_partials(part):
    return pl.pallas_call(
        _add_body,
        out_shape=jax.ShapeDtypeStruct((N_NODES, D_FEAT), jnp.float32),
        grid=(N_NODES // _ADD_ROWS,),
        in_specs=[
            pl.BlockSpec((1, _ADD_ROWS, D_FEAT), lambda i: (0, i, 0)),
            pl.BlockSpec((1, _ADD_ROWS, D_FEAT), lambda i: (1, i, 0)),
        ],
        out_specs=pl.BlockSpec((_ADD_ROWS, D_FEAT), lambda i: (i, 0)),
    )(part, part)


def kernel(x, edge_index):
    x1 = jnp.concatenate([x, jnp.zeros((1, D_FEAT), jnp.float32)])
    pad = E_PAD - N_EDGES
    src = jnp.concatenate(
        [edge_index[0], jnp.full((pad,), N_NODES, jnp.int32)])
    dst = jnp.concatenate([edge_index[1], jnp.zeros((pad,), jnp.int32)])
    idx = jnp.stack([src.reshape(NW, CHUNKS_PER_W, CHUNK),
                     dst.reshape(NW, CHUNKS_PER_W, CHUNK)], axis=2)
    zeros = jnp.zeros((ACC_ROWS, D_FEAT), jnp.float32)
    part = _sc_gather_scatter(x1, idx, zeros)
    return _combine_partials(part)


# speedup vs baseline: 2.9980x; 1.0933x over previous
import functools

import jax
import jax.numpy as jnp
from jax import lax
from jax.experimental import pallas as pl
from jax.experimental.pallas import tpu as pltpu
from jax.experimental.pallas import tpu_sc as plsc

N_NODES = 10000
D_FEAT = 128
N_EDGES = 320000

NC = 2
NS = 16
NW = NC * NS

CHUNK = 120
RING = 3
IR = 4
CHUNKS_PER_W = 84
ROUNDS = CHUNKS_PER_W // RING
PER_W = CHUNK * CHUNKS_PER_W
E_PAD = PER_W * NW

ACC_ROWS = N_NODES
ZERO_TILES = 10
ROWS_PER_ZT = ACC_ROWS // ZERO_TILES

_mesh = plsc.VectorSubcoreMesh(core_axis_name="c", subcore_axis_name="s")


@functools.partial(
    pl.kernel,
    mesh=_mesh,
    out_type=jax.ShapeDtypeStruct((NC, ACC_ROWS, D_FEAT), jnp.float32),
    scratch_types=[
        pltpu.VMEM_SHARED((ACC_ROWS, D_FEAT), jnp.float32),
        pltpu.VMEM((IR, 2, CHUNK), jnp.int32),
        pltpu.VMEM((RING, CHUNK, D_FEAT), jnp.float32),
        pltpu.SemaphoreType.DMA,
        pltpu.SemaphoreType.DMA,
        pltpu.SemaphoreType.DMA,
    ],
)
def _sc_gather_scatter(x_hbm, idx_hbm, zeros_hbm, part_hbm,
                       acc, idx_v, rows_v, isem, gsem, ssem):
    c = lax.axis_index("c")
    s = lax.axis_index("s")
    wid = s * NC + c

    for b in range(RING - 1):
        pltpu.sync_copy(idx_hbm.at[wid, b], idx_v.at[b])
        pltpu.async_copy(x_hbm.at[idx_v.at[b, 0]], rows_v.at[b], gsem)
    pltpu.async_copy(idx_hbm.at[wid, RING - 1], idx_v.at[RING - 1], isem)

    @pl.when(s < ZERO_TILES)
    def _():
        pltpu.sync_copy(zeros_hbm.at[pl.ds(s * ROWS_PER_ZT, ROWS_PER_ZT)],
                        acc.at[pl.ds(s * ROWS_PER_ZT, ROWS_PER_ZT)])
    plsc.subcore_barrier()

    def round_body(r, carry):
        for b in range(RING):
            g = r * RING + b
            pltpu.make_async_copy(x_hbm.at[idx_v.at[g % IR, 0]],
                                  rows_v.at[b], gsem).wait()
            pltpu.async_copy(rows_v.at[b], acc.at[idx_v.at[g % IR, 1]], ssem,
                             add=True)
            bp = (b - 1) % RING
            drain = pltpu.make_async_copy(
                rows_v.at[bp], acc.at[idx_v.at[(g - 1) % IR, 1]], ssem)
            if b == 0:
                @pl.when(r > 0)
                def _():
                    drain.wait()
            else:
                drain.wait()

            g2 = g + RING - 1
            g3 = g + RING

            def stage():
                pltpu.make_async_copy(idx_hbm.at[wid, g2],
                                      idx_v.at[g2 % IR], isem).wait()
                pltpu.async_copy(x_hbm.at[idx_v.at[g2 % IR, 0]],
                                 rows_v.at[bp], gsem)

            def prefetch():
                pltpu.async_copy(idx_hbm.at[wid, g3], idx_v.at[g3 % IR], isem)

            if (ROUNDS - 1) * RING + b + RING - 1 < CHUNKS_PER_W:
                stage()
            else:
                pl.when(g2 < CHUNKS_PER_W)(stage)
            if (ROUNDS - 1) * RING + b + RING < CHUNKS_PER_W:
                prefetch()
            else:
                pl.when(g3 < CHUNKS_PER_W)(prefetch)
        return carry

    lax.fori_loop(0, ROUNDS, round_body, 0)

    pltpu.make_async_copy(rows_v.at[RING - 1],
                          acc.at[idx_v.at[(CHUNKS_PER_W - 1) % IR, 1]],
                          ssem).wait()

    plsc.subcore_barrier()
    @pl.when(s < ZERO_TILES)
    def _():
        pltpu.sync_copy(acc.at[pl.ds(s * ROWS_PER_ZT, ROWS_PER_ZT)],
                        part_hbm.at[c, pl.ds(s * ROWS_PER_ZT, ROWS_PER_ZT)])


def _add_body(a_ref, b_ref, o_ref):
    o_ref[...] = a_ref[0] + b_ref[0]


_ADD_ROWS = 400


def _combine_partials(part):
    return pl.pallas_call(
        _add_body,
        out_shape=jax.ShapeDtypeStruct((N_NODES, D_FEAT), jnp.float32),
        grid=(N_NODES // _ADD_ROWS,),
        in_specs=[
            pl.BlockSpec((1, _ADD_ROWS, D_FEAT), lambda i: (0, i, 0)),
            pl.BlockSpec((1, _ADD_ROWS, D_FEAT), lambda i: (1, i, 0)),
        ],
        out_specs=pl.BlockSpec((_ADD_ROWS, D_FEAT), lambda i: (i, 0)),
    )(part, part)


def kernel(x, edge_index):
    x1 = jnp.concatenate([x, jnp.zeros((1, D_FEAT), jnp.float32)])
    pad = E_PAD - N_EDGES
    src = jnp.concatenate(
        [edge_index[0], jnp.full((pad,), N_NODES, jnp.int32)])
    dst = jnp.concatenate([edge_index[1], jnp.zeros((pad,), jnp.int32)])
    idx = jnp.stack([src.reshape(NW, CHUNKS_PER_W, CHUNK),
                     dst.reshape(NW, CHUNKS_PER_W, CHUNK)], axis=2)
    zeros = jnp.zeros((ACC_ROWS, D_FEAT), jnp.float32)
    part = _sc_gather_scatter(x1, idx, zeros)
    return _combine_partials(part)


# speedup vs baseline: 3.0019x; 1.0013x over previous
import functools

import jax
import jax.numpy as jnp
from jax import lax
from jax.experimental import pallas as pl
from jax.experimental.pallas import tpu as pltpu
from jax.experimental.pallas import tpu_sc as plsc

N_NODES = 10000
D_FEAT = 128
N_EDGES = 320000

NC = 2
NS = 16
NW = NC * NS

CHUNK = 120
RING = 3
IR = 4
CHUNKS_PER_W = 84
ROUNDS = CHUNKS_PER_W // RING
PER_W = CHUNK * CHUNKS_PER_W
E_PAD = PER_W * NW

ACC_ROWS = N_NODES
ZERO_TILES = 10
ROWS_PER_ZT = ACC_ROWS // ZERO_TILES

_mesh = plsc.VectorSubcoreMesh(core_axis_name="c", subcore_axis_name="s")


@functools.partial(
    pl.kernel,
    mesh=_mesh,
    out_type=jax.ShapeDtypeStruct((NC, ACC_ROWS, D_FEAT), jnp.float32),
    scratch_types=[
        pltpu.VMEM_SHARED((ACC_ROWS, D_FEAT), jnp.float32),
        pltpu.VMEM((IR, 2, CHUNK), jnp.int32),
        pltpu.VMEM((RING, CHUNK, D_FEAT), jnp.float32),
        pltpu.SemaphoreType.DMA,
        pltpu.SemaphoreType.DMA,
        pltpu.SemaphoreType.DMA,
    ],
)
def _sc_gather_scatter(x_hbm, idx_hbm, zeros_hbm, part_hbm,
                       acc, idx_v, rows_v, isem, gsem, ssem):
    c = lax.axis_index("c")
    s = lax.axis_index("s")
    wid = s * NC + c

    for b in range(RING - 1):
        pltpu.sync_copy(idx_hbm.at[wid, b], idx_v.at[b])
        pltpu.async_copy(x_hbm.at[idx_v.at[b, 0]], rows_v.at[b], gsem)
    pltpu.async_copy(idx_hbm.at[wid, RING - 1], idx_v.at[RING - 1], isem)

    @pl.when(s < ZERO_TILES)
    def _():
        pltpu.sync_copy(zeros_hbm.at[pl.ds(s * ROWS_PER_ZT, ROWS_PER_ZT)],
                        acc.at[pl.ds(s * ROWS_PER_ZT, ROWS_PER_ZT)])
    plsc.subcore_barrier()

    def round_body(r, carry):
        for b in range(RING):
            g = r * RING + b
            pltpu.make_async_copy(x_hbm.at[idx_v.at[g % IR, 0]],
                                  rows_v.at[b], gsem).wait()
            bp = (b - 1) % RING
            drain = pltpu.make_async_copy(
                rows_v.at[bp], acc.at[idx_v.at[(g - 1) % IR, 1]], ssem)
            if b == 0:
                @pl.when(r > 0)
                def _():
                    drain.wait()
            else:
                drain.wait()

            g2 = g + RING - 1
            g3 = g + RING

            def stage():
                pltpu.make_async_copy(idx_hbm.at[wid, g2],
                                      idx_v.at[g2 % IR], isem).wait()
                pltpu.async_copy(x_hbm.at[idx_v.at[g2 % IR, 0]],
                                 rows_v.at[bp], gsem)

            if (ROUNDS - 1) * RING + b + RING - 1 < CHUNKS_PER_W:
                stage()
            else:
                pl.when(g2 < CHUNKS_PER_W)(stage)

            pltpu.async_copy(rows_v.at[b], acc.at[idx_v.at[g % IR, 1]], ssem,
                             add=True)

            def prefetch():
                pltpu.async_copy(idx_hbm.at[wid, g3], idx_v.at[g3 % IR], isem)

            if (ROUNDS - 1) * RING + b + RING < CHUNKS_PER_W:
                prefetch()
            else:
                pl.when(g3 < CHUNKS_PER_W)(prefetch)
        return carry

    lax.fori_loop(0, ROUNDS, round_body, 0)

    pltpu.make_async_copy(rows_v.at[RING - 1],
                          acc.at[idx_v.at[(CHUNKS_PER_W - 1) % IR, 1]],
                          ssem).wait()

    plsc.subcore_barrier()
    @pl.when(s < ZERO_TILES)
    def _():
        pltpu.sync_copy(acc.at[pl.ds(s * ROWS_PER_ZT, ROWS_PER_ZT)],
                        part_hbm.at[c, pl.ds(s * ROWS_PER_ZT, ROWS_PER_ZT)])


def _add_body(a_ref, b_ref, o_ref):
    o_ref[...] = a_ref[0] + b_ref[0]


_ADD_ROWS = 400


def _combine_partials(part):
    return pl.pallas_call(
        _add_body,
        out_shape=jax.ShapeDtypeStruct((N_NODES, D_FEAT), jnp.float32),
        grid=(N_NODES // _ADD_ROWS,),
        in_specs=[
            pl.BlockSpec((1, _ADD_ROWS, D_FEAT), lambda i: (0, i, 0)),
            pl.BlockSpec((1, _ADD_ROWS, D_FEAT), lambda i: (1, i, 0)),
        ],
        out_specs=pl.BlockSpec((_ADD_ROWS, D_FEAT), lambda i: (i, 0)),
    )(part, part)


def kernel(x, edge_index):
    x1 = jnp.concatenate([x, jnp.zeros((1, D_FEAT), jnp.float32)])
    pad = E_PAD - N_EDGES
    src = jnp.concatenate(
        [edge_index[0], jnp.full((pad,), N_NODES, jnp.int32)])
    dst = jnp.concatenate([edge_index[1], jnp.zeros((pad,), jnp.int32)])
    idx = jnp.stack([src.reshape(NW, CHUNKS_PER_W, CHUNK),
                     dst.reshape(NW, CHUNKS_PER_W, CHUNK)], axis=2)
    zeros = jnp.zeros((ACC_ROWS, D_FEAT), jnp.float32)
    part = _sc_gather_scatter(x1, idx, zeros)
    return _combine_partials(part)


# speedup vs baseline: 5.6646x; 1.8870x over previous
import functools

import jax
import jax.numpy as jnp
from jax import lax
from jax.experimental import pallas as pl
from jax.experimental.pallas import tpu as pltpu
from jax.experimental.pallas import tpu_sc as plsc

N_NODES = 10000
D_FEAT = 128
N_EDGES = 320000

NC = 2
NS = 16
NW = NC * NS

CHUNK = 120
RING = 3
IR = 4
PER_W = N_EDGES // NW
FULL = PER_W // CHUNK
TAIL = PER_W - FULL * CHUNK

ACC_ROWS = 10008
ZERO_TILES = 9
ROWS_PER_ZT = ACC_ROWS // ZERO_TILES
OUT_TILES = 10
ROWS_PER_OT = N_NODES // OUT_TILES

_mesh = plsc.VectorSubcoreMesh(core_axis_name="c", subcore_axis_name="s")


@functools.partial(
    pl.kernel,
    mesh=_mesh,
    out_type=jax.ShapeDtypeStruct((NC, N_NODES, D_FEAT), jnp.float32),
    scratch_types=[
        pltpu.VMEM_SHARED((ACC_ROWS, D_FEAT), jnp.float32),
        pltpu.VMEM((IR, CHUNK), jnp.int32),
        pltpu.VMEM((IR, CHUNK), jnp.int32),
        pltpu.VMEM((RING, CHUNK, D_FEAT), jnp.float32),
        pltpu.VMEM((TAIL,), jnp.int32),
        pltpu.VMEM((TAIL,), jnp.int32),
        pltpu.SemaphoreType.DMA,
        pltpu.SemaphoreType.DMA,
        pltpu.SemaphoreType.DMA,
    ],
)
def _sc_gather_scatter(x_hbm, edge_hbm, zeros_hbm, part_hbm,
                       acc, src_v, dst_v, rows_v, tsrc_v, tdst_v,
                       isem, gsem, ssem):
    c = lax.axis_index("c")
    s = lax.axis_index("s")
    wid = s * NC + c
    base = wid * PER_W

    for b in range(RING - 1):
        pltpu.sync_copy(edge_hbm.at[pl.ds(base + b * CHUNK, CHUNK)],
                        src_v.at[b])
        pltpu.sync_copy(edge_hbm.at[pl.ds(N_EDGES + base + b * CHUNK, CHUNK)],
                        dst_v.at[b])
        pltpu.async_copy(x_hbm.at[src_v.at[b]], rows_v.at[b], gsem)
    pltpu.async_copy(
        edge_hbm.at[pl.ds(base + (RING - 1) * CHUNK, CHUNK)],
        src_v.at[RING - 1], isem)
    pltpu.async_copy(
        edge_hbm.at[pl.ds(N_EDGES + base + (RING - 1) * CHUNK, CHUNK)],
        dst_v.at[RING - 1], isem)

    @pl.when(s < ZERO_TILES)
    def _():
        pltpu.sync_copy(zeros_hbm.at[pl.ds(s * ROWS_PER_ZT, ROWS_PER_ZT)],
                        acc.at[pl.ds(s * ROWS_PER_ZT, ROWS_PER_ZT)])
    plsc.subcore_barrier()

    def body(g, carry):
        b = g % RING
        gp = jnp.maximum(g - 1, 0)
        g2 = g + RING - 1
        g2c = jnp.minimum(g2, FULL - 1)
        g3 = g + RING
        g3c = jnp.minimum(g3, FULL - 1)

        pltpu.make_async_copy(x_hbm.at[src_v.at[g % IR]],
                              rows_v.at[b], gsem).wait()
        drain = pltpu.make_async_copy(
            rows_v.at[gp % RING], acc.at[dst_v.at[gp % IR]], ssem)

        @pl.when(g > 0)
        def _():
            drain.wait()

        @pl.when(g2 < FULL)
        def _():
            pltpu.make_async_copy(
                edge_hbm.at[pl.ds(base + g2c * CHUNK, CHUNK)],
                src_v.at[g2c % IR], isem).wait()
            pltpu.make_async_copy(
                edge_hbm.at[pl.ds(N_EDGES + base + g2c * CHUNK, CHUNK)],
                dst_v.at[g2c % IR], isem).wait()
            pltpu.async_copy(x_hbm.at[src_v.at[g2c % IR]],
                             rows_v.at[g2c % RING], gsem)

        pltpu.async_copy(rows_v.at[b], acc.at[dst_v.at[g % IR]], ssem,
                         add=True)

        @pl.when(g3 < FULL)
        def _():
            pltpu.async_copy(
                edge_hbm.at[pl.ds(base + g3c * CHUNK, CHUNK)],
                src_v.at[g3c % IR], isem)
            pltpu.async_copy(
                edge_hbm.at[pl.ds(N_EDGES + base + g3c * CHUNK, CHUNK)],
                dst_v.at[g3c % IR], isem)
        return carry

    lax.fori_loop(0, FULL, body, 0)

    pltpu.make_async_copy(rows_v.at[(FULL - 1) % RING],
                          acc.at[dst_v.at[(FULL - 1) % IR]], ssem).wait()

    pltpu.sync_copy(edge_hbm.at[pl.ds(base + FULL * CHUNK, TAIL)], tsrc_v)
    pltpu.sync_copy(edge_hbm.at[pl.ds(N_EDGES + base + FULL * CHUNK, TAIL)], tdst_v)
    pltpu.async_copy(x_hbm.at[tsrc_v], rows_v.at[0, pl.ds(0, TAIL)],
                     gsem).wait()
    pltpu.sync_copy(rows_v.at[0, pl.ds(0, TAIL)], acc.at[tdst_v], add=True)

    plsc.subcore_barrier()
    @pl.when(s < OUT_TILES)
    def _():
        pltpu.sync_copy(acc.at[pl.ds(s * ROWS_PER_OT, ROWS_PER_OT)],
                        part_hbm.at[c, pl.ds(s * ROWS_PER_OT, ROWS_PER_OT)])


def _add_body(a_ref, b_ref, o_ref):
    o_ref[...] = a_ref[0] + b_ref[0]


_ADD_ROWS = 400


def _combine_partials(part):
    return pl.pallas_call(
        _add_body,
        out_shape=jax.ShapeDtypeStruct((N_NODES, D_FEAT), jnp.float32),
        grid=(N_NODES // _ADD_ROWS,),
        in_specs=[
            pl.BlockSpec((1, _ADD_ROWS, D_FEAT), lambda i: (0, i, 0)),
            pl.BlockSpec((1, _ADD_ROWS, D_FEAT), lambda i: (1, i, 0)),
        ],
        out_specs=pl.BlockSpec((_ADD_ROWS, D_FEAT), lambda i: (i, 0)),
    )(part, part)


def kernel(x, edge_index):
    zeros = jnp.zeros((ACC_ROWS, D_FEAT), jnp.float32)
    part = _sc_gather_scatter(x, edge_index.reshape(2 * N_EDGES), zeros)
    return _combine_partials(part)


# speedup vs baseline: 6.0412x; 1.0665x over previous
import functools

import jax
import jax.numpy as jnp
from jax import lax
from jax.experimental import pallas as pl
from jax.experimental.pallas import tpu as pltpu
from jax.experimental.pallas import tpu_sc as plsc

N_NODES = 10000
D_FEAT = 128
N_EDGES = 320000

NC = 2
NS = 16
NW = NC * NS

CHUNK = 120
RING = 3
IR = 4
PER_W = N_EDGES // NW
FULL = PER_W // CHUNK
TAIL = PER_W - FULL * CHUNK

ACC_ROWS = 10008
ZERO_TILES = 9
ROWS_PER_ZT = ACC_ROWS // ZERO_TILES
OUT_TILES = 10
ROWS_PER_OT = N_NODES // OUT_TILES

_mesh = plsc.VectorSubcoreMesh(core_axis_name="c", subcore_axis_name="s")


@functools.partial(
    pl.kernel,
    mesh=_mesh,
    out_type=jax.ShapeDtypeStruct((NC, N_NODES, D_FEAT), jnp.float32),
    scratch_types=[
        pltpu.VMEM_SHARED((ACC_ROWS, D_FEAT), jnp.float32),
        pltpu.VMEM((IR, CHUNK), jnp.int32),
        pltpu.VMEM((IR, CHUNK), jnp.int32),
        pltpu.VMEM((RING, CHUNK, D_FEAT), jnp.float32),
        pltpu.VMEM((TAIL,), jnp.int32),
        pltpu.VMEM((TAIL,), jnp.int32),
        pltpu.SemaphoreType.DMA,
        pltpu.SemaphoreType.DMA,
        pltpu.SemaphoreType.DMA,
    ],
)
def _sc_gather_scatter(x_hbm, edge_hbm, zeros_hbm, part_hbm,
                       acc, src_v, dst_v, rows_v, tsrc_v, tdst_v,
                       isem, gsem, ssem):
    c = lax.axis_index("c")
    s = lax.axis_index("s")
    wid = s * NC + c
    base = wid * PER_W

    for b in range(RING - 1):
        pltpu.sync_copy(edge_hbm.at[pl.ds(base + b * CHUNK, CHUNK)],
                        src_v.at[b])
        pltpu.sync_copy(edge_hbm.at[pl.ds(N_EDGES + base + b * CHUNK, CHUNK)],
                        dst_v.at[b])
        pltpu.async_copy(x_hbm.at[src_v.at[b]], rows_v.at[b], gsem)
    pltpu.async_copy(
        edge_hbm.at[pl.ds(base + (RING - 1) * CHUNK, CHUNK)],
        src_v.at[RING - 1], isem)
    pltpu.async_copy(
        edge_hbm.at[pl.ds(N_EDGES + base + (RING - 1) * CHUNK, CHUNK)],
        dst_v.at[RING - 1], isem)

    @pl.when(s < ZERO_TILES)
    def _():
        pltpu.sync_copy(zeros_hbm,
                        acc.at[pl.ds(s * ROWS_PER_ZT, ROWS_PER_ZT)])
    plsc.subcore_barrier()

    def body(g, carry):
        b = g % RING
        gp = jnp.maximum(g - 1, 0)
        g2 = g + RING - 1
        g2c = jnp.minimum(g2, FULL - 1)
        g3 = g + RING
        g3c = jnp.minimum(g3, FULL - 1)

        pltpu.make_async_copy(x_hbm.at[src_v.at[g % IR]],
                              rows_v.at[b], gsem).wait()
        drain = pltpu.make_async_copy(
            rows_v.at[gp % RING], acc.at[dst_v.at[gp % IR]], ssem)

        @pl.when(g > 0)
        def _():
            drain.wait()

        @pl.when(g2 < FULL)
        def _():
            pltpu.make_async_copy(
                edge_hbm.at[pl.ds(base + g2c * CHUNK, CHUNK)],
                src_v.at[g2c % IR], isem).wait()
            pltpu.make_async_copy(
                edge_hbm.at[pl.ds(N_EDGES + base + g2c * CHUNK, CHUNK)],
                dst_v.at[g2c % IR], isem).wait()
            pltpu.async_copy(x_hbm.at[src_v.at[g2c % IR]],
                             rows_v.at[g2c % RING], gsem)

        pltpu.async_copy(rows_v.at[b], acc.at[dst_v.at[g % IR]], ssem,
                         add=True)

        @pl.when(g3 < FULL)
        def _():
            pltpu.async_copy(
                edge_hbm.at[pl.ds(base + g3c * CHUNK, CHUNK)],
                src_v.at[g3c % IR], isem)
            pltpu.async_copy(
                edge_hbm.at[pl.ds(N_EDGES + base + g3c * CHUNK, CHUNK)],
                dst_v.at[g3c % IR], isem)
        return carry

    lax.fori_loop(0, FULL, body, 0)

    pltpu.make_async_copy(rows_v.at[(FULL - 1) % RING],
                          acc.at[dst_v.at[(FULL - 1) % IR]], ssem).wait()

    pltpu.sync_copy(edge_hbm.at[pl.ds(base + FULL * CHUNK, TAIL)], tsrc_v)
    pltpu.sync_copy(edge_hbm.at[pl.ds(N_EDGES + base + FULL * CHUNK, TAIL)], tdst_v)
    pltpu.async_copy(x_hbm.at[tsrc_v], rows_v.at[0, pl.ds(0, TAIL)],
                     gsem).wait()
    pltpu.sync_copy(rows_v.at[0, pl.ds(0, TAIL)], acc.at[tdst_v], add=True)

    plsc.subcore_barrier()
    @pl.when(s < OUT_TILES)
    def _():
        pltpu.sync_copy(acc.at[pl.ds(s * ROWS_PER_OT, ROWS_PER_OT)],
                        part_hbm.at[c, pl.ds(s * ROWS_PER_OT, ROWS_PER_OT)])


def _add_body(a_ref, b_ref, o_ref):
    o_ref[...] = a_ref[0] + b_ref[0]


_ADD_ROWS = 1000


def _combine_partials(part):
    return pl.pallas_call(
        _add_body,
        out_shape=jax.ShapeDtypeStruct((N_NODES, D_FEAT), jnp.float32),
        grid=(N_NODES // _ADD_ROWS,),
        in_specs=[
            pl.BlockSpec((1, _ADD_ROWS, D_FEAT), lambda i: (0, i, 0)),
            pl.BlockSpec((1, _ADD_ROWS, D_FEAT), lambda i: (1, i, 0)),
        ],
        out_specs=pl.BlockSpec((_ADD_ROWS, D_FEAT), lambda i: (i, 0)),
    )(part, part)


def kernel(x, edge_index):
    zeros = jnp.zeros((ROWS_PER_ZT, D_FEAT), jnp.float32)
    part = _sc_gather_scatter(x, edge_index.reshape(2 * N_EDGES), zeros)
    return _combine_partials(part)
